# baseline (speedup 1.0000x reference)
import functools

import jax
import jax.numpy as jnp
from jax import lax
from jax.experimental import pallas as pl
from jax.experimental.pallas import tpu as pltpu
from jax.experimental.pallas import tpu_sc as plsc

N = 100000
NUM_WORKERS = 32
PER_W = 3128
PER_W_PAD = 3136
LAST_BASE = N - PER_W
C = 128
NFULL = PER_W // C
TAIL = PER_W - NFULL * C
K = 4
NGROUPS = NFULL // K
NT = PER_W_PAD // 16
DIMS = (16, 32, 32, 48)
COLS = (0, 16, 48, 80)
DOUT = 128

_mesh = plsc.VectorSubcoreMesh(core_axis_name="c", subcore_axis_name="s")

_scratch = [pltpu.VMEM((PER_W_PAD, 4), jnp.int32)]
_scratch.extend(pltpu.VMEM((PER_W_PAD,), jnp.int32) for _ in range(4))
for _k in range(K):
    _scratch.extend(pltpu.VMEM((C, d), jnp.float32) for d in DIMS)
_scratch.extend(pltpu.VMEM((TAIL, d), jnp.float32) for d in DIMS)
_scratch.extend(pltpu.SemaphoreType.DMA for _ in range(K + 1))


@functools.partial(
    pl.kernel,
    out_type=jax.ShapeDtypeStruct((N, DOUT), jnp.float32),
    mesh=_mesh,
    scratch_types=_scratch,
    compiler_params=pltpu.CompilerParams(
        use_tc_tiling_on_sc=False, needs_layout_passes=False
    ),
)
def _sc_lookup(code_levels, t0, t1, t2, t3, out, *s):
    tables = (t0, t1, t2, t3)
    block = s[0]
    ivs = list(s[1:5])
    slots = [list(s[5 + 4 * k:9 + 4 * k]) for k in range(K)]
    trvs = list(s[5 + 4 * K:9 + 4 * K])
    sems = s[9 + 4 * K:9 + 5 * K]
    tsem = s[9 + 5 * K]

    wid = lax.axis_index("s") * 2 + lax.axis_index("c")
    base = jnp.minimum(wid * PER_W, LAST_BASE)

    pltpu.sync_copy(code_levels.at[pl.ds(base, PER_W), :],
                    block.at[pl.ds(0, PER_W), :])
    lanes = lax.iota(jnp.int32, 16)

    @pl.loop(0, NT)
    def _(i):
        rows = jnp.minimum(i * 16 + lanes, PER_W - 1)
        for l in range(4):
            col = jnp.full((16,), l, jnp.int32)
            vec = plsc.load_gather(block, [rows, col])
            ivs[l][pl.ds(i * 16, 16)] = vec

    def fire(off, n, rvs, sem):
        for l in range(4):
            pltpu.async_copy(tables[l].at[ivs[l].at[pl.ds(off, n)]], rvs[l], sem)

    def drain(off, n, rvs, sem):
        for l in range(4):
            pltpu.make_async_copy(
                tables[l].at[ivs[l].at[pl.ds(off, n)]], rvs[l], sem
            ).wait()

    def write_out(off, n, rvs):
        for l in range(4):
            pltpu.sync_copy(
                rvs[l], out.at[pl.ds(base + off, n), pl.ds(COLS[l], DIMS[l])]
            )

    for k in range(K):
        fire(k * C, C, slots[k], sems[k])
    fire(NFULL * C, TAIL, trvs, tsem)

    @pl.loop(0, NGROUPS)
    def _(g):
        for k in range(K):
            jj = g * K + k
            off = jj * C
            drain(off, C, slots[k], sems[k])
            write_out(off, C, slots[k])

            @pl.when(jj + K < NFULL)
            def _():
                fire(off + K * C, C, slots[k], sems[k])

    drain(NFULL * C, TAIL, trvs, tsem)
    write_out(NFULL * C, TAIL, trvs)


def kernel(code_levels, table_0, table_1, table_2, table_3):
    return _sc_lookup(code_levels, table_0, table_1, table_2, table_3)


# speedup vs baseline: 1.5160x; 1.5160x over previous
import functools

import jax
import jax.numpy as jnp
from jax import lax
from jax.experimental import pallas as pl
from jax.experimental.pallas import tpu as pltpu
from jax.experimental.pallas import tpu_sc as plsc

N = 100000
NUM_WORKERS = 32
PER_W = 3128
LAST_BASE = N - PER_W
C = 128
NFULL = PER_W // C
TAIL = PER_W - NFULL * C
K = 4
NGROUPS = NFULL // K
DIMS = (16, 32, 32, 48)
COLS = (0, 16, 48, 80)
DOUT = 128

_mesh = plsc.VectorSubcoreMesh(core_axis_name="c", subcore_axis_name="s")

_scratch = [pltpu.VMEM((4, PER_W), jnp.int32)]
for _k in range(K):
    _scratch.extend(pltpu.VMEM((C, d), jnp.float32) for d in DIMS)
_scratch.extend(pltpu.VMEM((TAIL, d), jnp.float32) for d in DIMS)
_scratch.extend(pltpu.SemaphoreType.DMA for _ in range(K + 1))


@functools.partial(
    pl.kernel,
    out_type=jax.ShapeDtypeStruct((N, DOUT), jnp.float32),
    mesh=_mesh,
    scratch_types=_scratch,
    compiler_params=pltpu.CompilerParams(use_tc_tiling_on_sc=False),
)
def _sc_lookup(idx0, idx1, idx2, idx3, t0, t1, t2, t3, out, *s):
    idxs = (idx0, idx1, idx2, idx3)
    tables = (t0, t1, t2, t3)
    iv = s[0]
    slots = [list(s[1 + 4 * k:5 + 4 * k]) for k in range(K)]
    trvs = list(s[1 + 4 * K:5 + 4 * K])
    sems = s[5 + 4 * K:5 + 5 * K]
    tsem = s[5 + 5 * K]

    wid = lax.axis_index("s") * 2 + lax.axis_index("c")
    base = jnp.minimum(wid * PER_W, LAST_BASE)

    for l in range(4):
        pltpu.sync_copy(idxs[l].at[pl.ds(base, PER_W)], iv.at[l])

    def fire(off, n, rvs, sem):
        for l in range(4):
            pltpu.async_copy(tables[l].at[iv.at[l, pl.ds(off, n)]], rvs[l], sem)

    def drain(off, n, rvs, sem):
        for l in range(4):
            pltpu.make_async_copy(
                tables[l].at[iv.at[l, pl.ds(off, n)]], rvs[l], sem
            ).wait()

    def write_out(off, n, rvs):
        for l in range(4):
            pltpu.sync_copy(
                rvs[l], out.at[pl.ds(base + off, n), pl.ds(COLS[l], DIMS[l])]
            )

    for k in range(K):
        fire(k * C, C, slots[k], sems[k])
    fire(NFULL * C, TAIL, trvs, tsem)

    @pl.loop(0, NGROUPS)
    def _(g):
        for k in range(K):
            jj = g * K + k
            off = jj * C
            drain(off, C, slots[k], sems[k])
            write_out(off, C, slots[k])

            @pl.when(jj + K < NFULL)
            def _():
                fire(off + K * C, C, slots[k], sems[k])

    drain(NFULL * C, TAIL, trvs, tsem)
    write_out(NFULL * C, TAIL, trvs)


def kernel(code_levels, table_0, table_1, table_2, table_3):
    return _sc_lookup(
        code_levels[:, 0], code_levels[:, 1],
        code_levels[:, 2], code_levels[:, 3],
        table_0, table_1, table_2, table_3,
    )


# speedup vs baseline: 1.5422x; 1.0173x over previous
import functools

import jax
import jax.numpy as jnp
from jax import lax
from jax.experimental import pallas as pl
from jax.experimental.pallas import tpu as pltpu
from jax.experimental.pallas import tpu_sc as plsc

N = 100000
NUM_WORKERS = 32
PER_W = 3128
LAST_BASE = N - PER_W
C = 256
G = 128
NFULL = PER_W // C
TAIL = PER_W - NFULL * C
K = 3
NGROUPS = NFULL // K
DIMS = (16, 32, 32, 48)
COLS = (0, 16, 48, 80)
DOUT = 128

_mesh = plsc.VectorSubcoreMesh(core_axis_name="c", subcore_axis_name="s")

_scratch = [pltpu.VMEM((4, PER_W), jnp.int32)]
for _k in range(K):
    _scratch.extend(pltpu.VMEM((C, d), jnp.float32) for d in DIMS)
_scratch.extend(pltpu.VMEM((TAIL, d), jnp.float32) for d in DIMS)
_scratch.extend(pltpu.SemaphoreType.DMA for _ in range(K + 1))


@functools.partial(
    pl.kernel,
    out_type=jax.ShapeDtypeStruct((N, DOUT), jnp.float32),
    mesh=_mesh,
    scratch_types=_scratch,
    compiler_params=pltpu.CompilerParams(use_tc_tiling_on_sc=False),
)
def _sc_lookup(idx0, idx1, idx2, idx3, t0, t1, t2, t3, out, *s):
    idxs = (idx0, idx1, idx2, idx3)
    tables = (t0, t1, t2, t3)
    iv = s[0]
    slots = [list(s[1 + 4 * k:5 + 4 * k]) for k in range(K)]
    trvs = list(s[1 + 4 * K:5 + 4 * K])
    sems = s[5 + 4 * K:5 + 5 * K]
    tsem = s[5 + 5 * K]

    wid = lax.axis_index("s") * 2 + lax.axis_index("c")
    base = jnp.minimum(wid * PER_W, LAST_BASE)

    for l in range(4):
        pltpu.sync_copy(idxs[l].at[pl.ds(base, PER_W)], iv.at[l])

    def fire(off, n, rvs, sem):
        for l in range(4):
            for g0 in range(0, n, G):
                gl = min(G, n - g0)
                pltpu.async_copy(
                    tables[l].at[iv.at[l, pl.ds(off + g0, gl)]],
                    rvs[l].at[pl.ds(g0, gl)],
                    sem,
                )

    def drain(off, n, rvs, sem):
        for l in range(4):
            for g0 in range(0, n, G):
                gl = min(G, n - g0)
                pltpu.make_async_copy(
                    tables[l].at[iv.at[l, pl.ds(off + g0, gl)]],
                    rvs[l].at[pl.ds(g0, gl)],
                    sem,
                ).wait()

    def write_out(off, n, rvs):
        for l in range(4):
            pltpu.sync_copy(
                rvs[l], out.at[pl.ds(base + off, n), pl.ds(COLS[l], DIMS[l])]
            )

    for k in range(K):
        fire(k * C, C, slots[k], sems[k])
    fire(NFULL * C, TAIL, trvs, tsem)

    @pl.loop(0, NGROUPS)
    def _(g):
        for k in range(K):
            jj = g * K + k
            off = jj * C
            drain(off, C, slots[k], sems[k])
            write_out(off, C, slots[k])

            @pl.when(jj + K < NFULL)
            def _():
                fire(off + K * C, C, slots[k], sems[k])

    drain(NFULL * C, TAIL, trvs, tsem)
    write_out(NFULL * C, TAIL, trvs)


def kernel(code_levels, table_0, table_1, table_2, table_3):
    return _sc_lookup(
        code_levels[:, 0], code_levels[:, 1],
        code_levels[:, 2], code_levels[:, 3],
        table_0, table_1, table_2, table_3,
    )
